# Initial kernel scaffold; baseline (speedup 1.0000x reference)
import jax
import jax.numpy as jnp
from jax.experimental import pallas as pl


def kernel(inputs, embedding):
    raise NotImplementedError("write your pallas kernel here")


# speedup vs baseline: 3.3482x; 3.3482x over previous
import functools

import jax
import jax.numpy as jnp
from jax import lax
from jax.experimental import pallas as pl
from jax.experimental.pallas import tpu as pltpu
from jax.experimental.pallas import tpu_sc as plsc

VOCAB_SIZE = 100000
EMBED_DIM = 128

NC = 2
NS = 16
NW = NC * NS

IDX_W = 128
NB = 5


def _make_gather(B):
    assert B % (NW * IDX_W) == 0
    n_chunk = B // (NW * IDX_W)
    assert n_chunk % NB == 0
    n_iter = n_chunk // NB

    mesh = plsc.VectorSubcoreMesh(core_axis_name="c", subcore_axis_name="s")

    @functools.partial(
        pl.kernel,
        mesh=mesh,
        out_type=jax.ShapeDtypeStruct((B, EMBED_DIM), jnp.float32),
        scratch_types=[
            pltpu.VMEM((n_chunk, IDX_W), jnp.int32),
            pltpu.VMEM((NB, IDX_W, EMBED_DIM), jnp.float32),
            [pltpu.SemaphoreType.DMA] * NB,
            [pltpu.SemaphoreType.DMA] * NB,
        ],
    )
    def gather_kernel(table_hbm, idx_hbm, out_hbm, idx_v, rows_v, gsems, wsems):
        wid = lax.axis_index("s") * NC + lax.axis_index("c")
        row_base = wid * (n_chunk * IDX_W)

        pltpu.sync_copy(idx_hbm.at[wid], idx_v)

        def start_gather(j, b):
            pltpu.async_copy(table_hbm.at[idx_v.at[j]], rows_v.at[b], gsems[b])

        def wait_gather(b):
            pltpu.make_async_copy(
                table_hbm.at[idx_v.at[0]], rows_v.at[b], gsems[b]
            ).wait()

        def start_write(j, b):
            pltpu.async_copy(
                rows_v.at[b],
                out_hbm.at[pl.ds(row_base + j * IDX_W, IDX_W)],
                wsems[b],
            )

        def wait_write(b):
            pltpu.make_async_copy(
                rows_v.at[b], out_hbm.at[pl.ds(0, IDX_W)], wsems[b]
            ).wait()

        for b in range(NB):
            start_gather(b, b)

        def body(i, _):
            for b in range(NB):
                j = i * NB + b
                wait_gather(b)
                start_write(j, b)
                wait_write(b)
                start_gather(j + NB, b)
            return ()

        if n_iter > 1:
            lax.fori_loop(0, n_iter - 1, body, (), unroll=False)

        for b in range(NB):
            j = (n_iter - 1) * NB + b
            wait_gather(b)
            start_write(j, b)
        for b in range(NB):
            wait_write(b)

    return gather_kernel


def kernel(inputs, embedding):
    B = inputs.size
    idx = inputs.reshape(-1).astype(jnp.int32)
    idx = idx.reshape(NW, B // (NW * IDX_W), IDX_W)
    out = _make_gather(B)(embedding, idx)
    return out.reshape(*inputs.shape, EMBED_DIM)
